# baseline (speedup 1.0000x reference)
import functools

import jax
import jax.numpy as jnp
from jax import lax
from jax.experimental import pallas as pl
from jax.experimental.pallas import tpu as pltpu
from jax.experimental.pallas import tpu_sc as plsc

_EMBED = 64
_NC = 2
_NS = 16
_NW = _NC * _NS
_IDXW = 128
_CHUNK = 512
_SUB = _CHUNK // _IDXW


@functools.lru_cache(maxsize=None)
def _build(rows_total: int):
  b_per_w = rows_total // _NW
  n_idx_rows = b_per_w // _IDXW
  n_pairs = (b_per_w // _CHUNK) // 2
  mesh = plsc.VectorSubcoreMesh(
      core_axis_name="c", subcore_axis_name="s",
      num_cores=_NC, num_subcores=_NS)

  @functools.partial(
      pl.kernel,
      out_type=jax.ShapeDtypeStruct((rows_total, _EMBED), jnp.float32),
      mesh=mesh,
      compiler_params=pltpu.CompilerParams(
          needs_layout_passes=False, use_tc_tiling_on_sc=False),
      scratch_types=[
          pltpu.VMEM((n_idx_rows, _IDXW), jnp.int32),
          pltpu.VMEM((_CHUNK, _EMBED), jnp.float32),
          pltpu.VMEM((_CHUNK, _EMBED), jnp.float32),
          pltpu.SemaphoreType.DMA,
          pltpu.SemaphoreType.DMA,
          pltpu.SemaphoreType.DMA,
      ],
  )
  def emb(table_hbm, idx_hbm, out_hbm, idx_v, rows0, rows1, gsem, osem0,
          osem1):
    wid = lax.axis_index("s") * _NC + lax.axis_index("c")
    out_base = wid * b_per_w
    pltpu.sync_copy(idx_hbm.at[pl.ds(wid * n_idx_rows, n_idx_rows), :], idx_v)

    bufs = (rows0, rows1)
    osems = (osem0, osem1)

    def run_chunk(g, rows_buf, osem, not_first):
      @pl.when(not_first)
      def _():
        pltpu.make_async_copy(
            rows_buf, out_hbm.at[pl.ds(0, _CHUNK), :], osem).wait()

      cps = []
      for s in range(_SUB):
        cps.append(pltpu.async_copy(
            table_hbm.at[idx_v.at[g * _SUB + s]],
            rows_buf.at[pl.ds(s * _IDXW, _IDXW), :],
            gsem))

      any_zero = None
      for s in range(_SUB):
        for i in range(_IDXW // 16):
          v = idx_v[g * _SUB + s, pl.ds(i * 16, 16)]
          zm = v == 0
          any_zero = zm if any_zero is None else (any_zero | zm)

      for cp in cps:
        cp.wait()

      @pl.when(plsc.all_reduce_population_count(any_zero)[0] > 0)
      def _():
        def fix_group(gi, carry):
          s = gi // (_IDXW // 16)
          i = gi - s * (_IDXW // 16)
          v = idx_v[g * _SUB + s, pl.ds(i * 16, 16)]
          zm = v == 0
          rowids = (s * _IDXW + i * 16) + lax.iota(jnp.int32, 16)

          @pl.when(plsc.all_reduce_population_count(zm)[0] > 0)
          def _():
            def fix_col(col, inner):
              plsc.store_scatter(
                  rows_buf,
                  [rowids, jnp.zeros((16,), jnp.int32) + col],
                  jnp.zeros((16,), jnp.float32),
                  mask=zm)
              return inner
            lax.fori_loop(0, _EMBED, fix_col, 0)
          return carry
        lax.fori_loop(0, _CHUNK // 16, fix_group, 0)

      pltpu.async_copy(
          rows_buf, out_hbm.at[pl.ds(out_base + g * _CHUNK, _CHUNK), :], osem)

    @pl.loop(0, n_pairs)
    def _pairs(g2):
      for p in range(2):
        run_chunk(g2 * 2 + p, bufs[p], osems[p], g2 > 0)

    for p in range(2):
      pltpu.make_async_copy(
          bufs[p], out_hbm.at[pl.ds(0, _CHUNK), :], osems[p]).wait()

  return emb


@jax.jit
def kernel(table, input):
  b, s = input.shape
  rows_total = b * s
  idx = input.reshape(rows_total // _IDXW, _IDXW).astype(jnp.int32)
  out = _build(rows_total)(table, idx)
  return out.reshape(b, s, _EMBED)


# speedup vs baseline: 1.2161x; 1.2161x over previous
import functools

import jax
import jax.numpy as jnp
from jax import lax
from jax.experimental import pallas as pl
from jax.experimental.pallas import tpu as pltpu
from jax.experimental.pallas import tpu_sc as plsc

_EMBED = 64
_NC = 2
_NS = 16
_NW = _NC * _NS
_IDXW = 128
_CHUNK = 256
_SUB = _CHUNK // _IDXW


@functools.lru_cache(maxsize=None)
def _build(rows_total: int):
  b_per_w = rows_total // _NW
  n_idx_rows = b_per_w // _IDXW
  n_pairs = (b_per_w // _CHUNK) // 2
  mesh = plsc.VectorSubcoreMesh(
      core_axis_name="c", subcore_axis_name="s",
      num_cores=_NC, num_subcores=_NS)

  @functools.partial(
      pl.kernel,
      out_type=jax.ShapeDtypeStruct((rows_total, 2 * _EMBED), jnp.float32),
      mesh=mesh,
      compiler_params=pltpu.CompilerParams(
          needs_layout_passes=False, use_tc_tiling_on_sc=True),
      scratch_types=[
          pltpu.VMEM((n_idx_rows, _IDXW), jnp.int32),
          pltpu.VMEM((_CHUNK, 2 * _EMBED), jnp.float32),
          pltpu.VMEM((_CHUNK, 2 * _EMBED), jnp.float32),
          pltpu.SemaphoreType.DMA,
          pltpu.SemaphoreType.DMA,
          pltpu.SemaphoreType.DMA,
      ],
  )
  def emb(table_hbm, idx_hbm, out_hbm, idx_v, rows0, rows1, gsem, osem0,
          osem1):
    wid = lax.axis_index("s") * _NC + lax.axis_index("c")
    out_base = wid * b_per_w
    pltpu.sync_copy(idx_hbm.at[wid], idx_v)

    bufs = (rows0, rows1)
    osems = (osem0, osem1)

    def run_chunk(g, rows_buf, osem, not_first):
      @pl.when(not_first)
      def _():
        pltpu.make_async_copy(
            rows_buf, out_hbm.at[pl.ds(0, _CHUNK), :], osem).wait()

      cps = []
      for s in range(_SUB):
        cps.append(pltpu.async_copy(
            table_hbm.at[idx_v.at[g * _SUB + s]],
            rows_buf.at[pl.ds(s * _IDXW, _IDXW), :],
            gsem))

      any_zero = None
      for s in range(_SUB):
        for i in range(_IDXW // 16):
          v = idx_v[g * _SUB + s, pl.ds(i * 16, 16)]
          zm = v == 0
          any_zero = zm if any_zero is None else (any_zero | zm)

      for cp in cps:
        cp.wait()

      @pl.when(plsc.all_reduce_population_count(any_zero)[0] > 0)
      def _():
        def fix_group(gi, carry):
          s = gi // (_IDXW // 16)
          i = gi - s * (_IDXW // 16)
          v = idx_v[g * _SUB + s, pl.ds(i * 16, 16)]
          zm = v == 0
          rowids = (s * _IDXW + i * 16) + lax.iota(jnp.int32, 16)

          @pl.when(plsc.all_reduce_population_count(zm)[0] > 0)
          def _():
            def fix_col(col, inner):
              plsc.store_scatter(
                  rows_buf,
                  [rowids, jnp.zeros((16,), jnp.int32) + col],
                  jnp.zeros((16,), jnp.float32),
                  mask=zm)
              return inner
            lax.fori_loop(0, _EMBED, fix_col, 0)
          return carry
        lax.fori_loop(0, _CHUNK // 16, fix_group, 0)

      pltpu.async_copy(
          rows_buf, out_hbm.at[pl.ds(out_base + g * _CHUNK, _CHUNK), :], osem)

    @pl.loop(0, n_pairs)
    def _pairs(g2):
      for p in range(2):
        run_chunk(g2 * 2 + p, bufs[p], osems[p], g2 > 0)

    for p in range(2):
      pltpu.make_async_copy(
          bufs[p], out_hbm.at[pl.ds(0, _CHUNK), :], osems[p]).wait()

  return emb


@jax.jit
def kernel(table, input):
  b, s = input.shape
  rows_total = b * s
  table_p = jnp.pad(table, ((0, 0), (0, 2 * _EMBED - table.shape[1])))
  idx = input.reshape(_NW, rows_total // (_NW * _IDXW), _IDXW).astype(
      jnp.int32)
  out = _build(rows_total)(table_p, idx)
  return out[:, :_EMBED].reshape(b, s, _EMBED)
